# Initial kernel scaffold; baseline (speedup 1.0000x reference)
import jax
import jax.numpy as jnp
from jax.experimental import pallas as pl


def kernel(ids, embed_weight):
    raise NotImplementedError("write your pallas kernel here")


# speedup vs baseline: 10.3793x; 10.3793x over previous
import jax
import jax.numpy as jnp
from jax import lax
from jax.experimental import pallas as pl
from jax.experimental.pallas import tpu as pltpu

N_REGIONS = 64
TEXT_LEN = 512
VOCAB = 256
D_MODEL = 128


def _body(ids_t_ref, w_ref, out_ref):
    vocab_col = lax.broadcasted_iota(jnp.int32, (VOCAB, N_REGIONS), 0)

    def step(b, acc):
        blk = ids_t_ref[pl.ds(b * 8, 8), :]
        for s in range(8):
            row = blk[s:s + 1, :]
            acc = acc + (row == vocab_col).astype(jnp.float32)
        return acc

    counts_t = lax.fori_loop(0, TEXT_LEN // 8, step,
                             jnp.zeros((VOCAB, N_REGIONS), jnp.float32))
    out_ref[...] = lax.dot_general(
        counts_t, w_ref[...], (((0,), (0,)), ((), ())),
        preferred_element_type=jnp.float32) * (1.0 / TEXT_LEN)


def kernel(ids, embed_weight):
    ids_t = ids.T
    return pl.pallas_call(
        _body,
        out_shape=jax.ShapeDtypeStruct((N_REGIONS, D_MODEL), jnp.float32),
        in_specs=[
            pl.BlockSpec(memory_space=pltpu.VMEM),
            pl.BlockSpec(memory_space=pltpu.VMEM),
        ],
        out_specs=pl.BlockSpec(memory_space=pltpu.VMEM),
    )(ids_t, embed_weight)
